# Initial kernel scaffold; baseline (speedup 1.0000x reference)
import jax
import jax.numpy as jnp
from jax.experimental import pallas as pl


def kernel(inputs, lookup_table_real, lookup_table_imag):
    raise NotImplementedError("write your pallas kernel here")


# speedup vs baseline: 6.0886x; 6.0886x over previous
import functools

import jax
import jax.numpy as jnp
from jax.experimental import pallas as pl
from jax.experimental.pallas import tpu as pltpu


def _fe_block(x_ref, wr_ref, wi_ref, o_ref):
    x = x_ref[...]
    wr = wr_ref[...]
    wi = wi_ref[...]
    real = jnp.dot(x[:, :128], wr, preferred_element_type=jnp.float32)
    imag = jnp.dot(x[:, 128:], wi, preferred_element_type=jnp.float32)
    o_ref[...] = jnp.concatenate([real, imag], axis=-1)


def kernel(inputs, lookup_table_real, lookup_table_imag):
    B, T, F2 = inputs.shape
    D = lookup_table_real.shape[1] * 2
    rows = B * T
    x = inputs.reshape(rows, F2)

    R = 2048
    assert rows % R == 0
    grid = (rows // R,)

    out = pl.pallas_call(
        _fe_block,
        grid=grid,
        in_specs=[
            pl.BlockSpec((R, F2), lambda i: (i, 0)),
            pl.BlockSpec((128, D // 2), lambda i: (0, 0)),
            pl.BlockSpec((128, D // 2), lambda i: (0, 0)),
        ],
        out_specs=pl.BlockSpec((R, D), lambda i: (i, 0)),
        out_shape=jax.ShapeDtypeStruct((rows, D), jnp.float32),
    )(x, lookup_table_real, lookup_table_imag)
    return out.reshape(B, T, D)


# speedup vs baseline: 7.3584x; 1.2086x over previous
import jax
import jax.numpy as jnp
from jax.experimental import pallas as pl


def _fe_block(x_ref, w_ref, o_ref):
    o_ref[...] = jnp.dot(x_ref[...], w_ref[...],
                         preferred_element_type=jnp.float32)


def kernel(inputs, lookup_table_real, lookup_table_imag):
    B, T, F2 = inputs.shape
    half = lookup_table_real.shape[1]
    D = 2 * half
    F = F2 // 2
    rows = B * T
    x = inputs.reshape(rows, F2)

    w = jnp.zeros((F2, D), jnp.float32)
    w = w.at[:F, :half].set(lookup_table_real)
    w = w.at[F:, half:].set(lookup_table_imag)

    R = 4096
    assert rows % R == 0
    grid = (rows // R,)

    out = pl.pallas_call(
        _fe_block,
        grid=grid,
        in_specs=[
            pl.BlockSpec((R, F2), lambda i: (i, 0)),
            pl.BlockSpec((F2, D), lambda i: (0, 0)),
        ],
        out_specs=pl.BlockSpec((R, D), lambda i: (i, 0)),
        out_shape=jax.ShapeDtypeStruct((rows, D), jnp.float32),
    )(x, w)
    return out.reshape(B, T, D)
